# baseline (speedup 1.0000x reference)
import functools

import jax
import jax.numpy as jnp
from jax import lax
from jax.experimental import pallas as pl
from jax.experimental.pallas import tpu as pltpu
from jax.experimental.pallas import tpu_sc as plsc

_NC = 2
_NS = 16
_LANES = 16


def _make_edge_kernel(n_nodes, n_edges, heads, ch):
    hc = heads * ch
    w = hc + 16
    nw = _NC * _NS
    ew = n_edges // nw
    assert ew * nw == n_edges
    chunk = 40
    nchunk = ew // chunk
    assert nchunk * chunk == ew
    rpt = n_nodes // _NS
    assert rpt * _NS == n_nodes
    groups = hc // _LANES
    hpg = _LANES // ch

    assert nchunk % 2 == 0 and nchunk >= 4
    mesh = plsc.VectorSubcoreMesh(core_axis_name="c", subcore_axis_name="s")

    def buf_scratch():
        return [
            pltpu.VMEM((chunk,), jnp.int32),
            pltpu.VMEM((chunk,), jnp.int32),
            pltpu.VMEM((chunk, w), jnp.float32),
            pltpu.VMEM((chunk, 16), jnp.float32),
            pltpu.VMEM((chunk, w), jnp.float32),
            pltpu.SemaphoreType.DMA,
            pltpu.SemaphoreType.DMA,
        ]

    @functools.partial(
        pl.kernel,
        out_type=jax.ShapeDtypeStruct((_NC, n_nodes, w), jnp.float32),
        mesh=mesh,
        compiler_params=pltpu.CompilerParams(use_tc_tiling_on_sc=False,
                                             needs_layout_passes=False),
        scratch_types=buf_scratch() + buf_scratch() + [
            pltpu.VMEM_SHARED((n_nodes, w), jnp.float32),
        ],
    )
    def edge_kernel(hsrc_hbm, ad_hbm, src_hbm, dst_hbm, zeros_hbm, out_hbm,
                    *scratch):
        bufs = (scratch[0:7], scratch[7:14])
        acc = scratch[14]
        c = lax.axis_index("c")
        s = lax.axis_index("s")
        wid = c * _NS + s
        pltpu.sync_copy(zeros_hbm.at[pl.ds(s * rpt, rpt)],
                        acc.at[pl.ds(s * rpt, rpt)])
        plsc.subcore_barrier()

        iota = lax.iota(jnp.int32, _LANES)
        gdn = lax.GatherDimensionNumbers(
            offset_dims=(), collapsed_slice_dims=(0,), start_index_map=(0,))

        def issue(ci, buf):
            sidx, didx, hrows, adrows, _, semg, _2 = buf
            base = wid * ew + ci * chunk
            pltpu.sync_copy(src_hbm.at[pl.ds(base, chunk)], sidx)
            pltpu.sync_copy(dst_hbm.at[pl.ds(base, chunk)], didx)
            pltpu.async_copy(hsrc_hbm.at[sidx], hrows, semg)
            pltpu.async_copy(ad_hbm.at[didx], adrows, semg)

        def scatter_start(buf):
            _, didx, _2, _3, msg, _4, sems = buf
            pltpu.async_copy(msg, acc.at[didx], sems, add=True)

        def scatter_wait(buf):
            _, didx, _2, _3, msg, _4, sems = buf
            pltpu.make_async_copy(msg, acc.at[didx], sems).wait()

        def compute(buf):
            sidx, didx, hrows, adrows, msg, semg, _ = buf
            pltpu.make_async_copy(hsrc_hbm.at[sidx], hrows, semg).wait()
            pltpu.make_async_copy(ad_hbm.at[didx], adrows, semg).wait()

            @plsc.parallel_loop(0, chunk, unroll=8)
            def edge_body(e):
                va = hrows[e, pl.ds(hc, 16)]
                vd = adrows[e, pl.ds(0, 16)]
                logit = va + vd
                logit = jnp.where(logit > 0, logit, 0.2 * logit)
                t = jnp.exp(logit)
                msg[e, pl.ds(hc, 16)] = t
                for g in range(groups):
                    gidx = g * hpg + iota // ch
                    mult = lax.gather(t, gidx[:, None], gdn, (1,),
                                      mode=lax.GatherScatterMode.PROMISE_IN_BOUNDS)
                    msg[e, pl.ds(g * _LANES, _LANES)] = (
                        hrows[e, pl.ds(g * _LANES, _LANES)] * mult)
        issue(0, bufs[0])
        issue(1, bufs[1])

        def pair_body(j, carry):
            c0 = 2 * j
            compute(bufs[0])
            scatter_start(bufs[0])
            compute(bufs[1])
            scatter_wait(bufs[0])
            issue(c0 + 2, bufs[0])
            scatter_start(bufs[1])
            scatter_wait(bufs[1])
            issue(c0 + 3, bufs[1])
            return carry

        lax.fori_loop(0, nchunk // 2 - 1, pair_body, 0)
        compute(bufs[0])
        scatter_start(bufs[0])
        compute(bufs[1])
        scatter_wait(bufs[0])
        scatter_start(bufs[1])
        scatter_wait(bufs[1])
        plsc.subcore_barrier()
        pltpu.sync_copy(acc.at[pl.ds(s * rpt, rpt)],
                        out_hbm.at[c, pl.ds(s * rpt, rpt)])

    return edge_kernel


def _tc1_body(x_ref, gb_ref, w1e_ref, w1a_ref, h_out, ad_out):
    x = x_ref[...]
    mean = jnp.mean(x, axis=0, keepdims=True)
    xc = x - mean
    var = jnp.mean(xc * xc, axis=0, keepdims=True)
    xh = xc / jnp.sqrt(var + 1e-5) * gb_ref[0:1, :] + gb_ref[1:2, :]
    h_out[...] = jnp.dot(xh, w1e_ref[...], preferred_element_type=jnp.float32)
    ad_out[...] = jnp.dot(xh, w1a_ref[...], preferred_element_type=jnp.float32)


def _tc2_body(acc_ref, b1_ref, p1_ref, w2e_ref, w2a_ref, h_out, ad_out):
    a = acc_ref[0] + acc_ref[1]
    num = a[:, :64]
    den = jnp.dot(a, p1_ref[...], preferred_element_type=jnp.float32,
                  precision=lax.Precision.HIGHEST)
    o = num / (den + 1e-16) + b1_ref[...]
    h = jnp.where(o > 0, o, jnp.exp(jnp.minimum(o, 0.0)) - 1.0)
    h_out[...] = jnp.dot(h, w2e_ref[...], preferred_element_type=jnp.float32)
    ad_out[...] = jnp.dot(h, w2a_ref[...], preferred_element_type=jnp.float32)


def _tc3_body(acc_ref, b2_ref, p2_ref, s_ref, out_ref):
    a = acc_ref[0] + acc_ref[1]
    num = a[:, :128]
    den = jnp.dot(a, p2_ref[...], preferred_element_type=jnp.float32,
                  precision=lax.Precision.HIGHEST)
    ratio = num / (den + 1e-16)
    o = jnp.dot(ratio, s_ref[...], preferred_element_type=jnp.float32,
                precision=lax.Precision.HIGHEST) * 0.125 + b2_ref[...]
    m = jnp.max(o, axis=1, keepdims=True)
    z = o - m
    lse = jnp.log(jnp.sum(jnp.exp(z), axis=1, keepdims=True))
    out_ref[...] = z - lse


def _head_proj(a_vec, heads, ch):
    return (a_vec[:, :, None] * jnp.eye(heads, dtype=a_vec.dtype)[:, None, :]
            ).reshape(heads * ch, heads)


def kernel(x, edge_index, gamma, beta, W1, a_src1, a_dst1, b1,
           W2, a_src2, a_dst2, b2):
    n, d = x.shape
    e = edge_index.shape[1]
    f32 = jnp.float32
    hi = lax.Precision.HIGHEST

    src = edge_index[0]
    dst = edge_index[1]

    as1 = _head_proj(a_src1, 8, 8)
    ad1 = _head_proj(a_dst1, 8, 8)
    as2 = _head_proj(a_src2, 8, 16)
    ad2 = _head_proj(a_dst2, 8, 16)
    z8_64 = jnp.zeros((d, 8), f32)
    w1e = jnp.concatenate([W1, jnp.dot(W1, as1, precision=hi), z8_64], axis=1)
    w1a = jnp.concatenate([jnp.dot(W1, ad1, precision=hi), z8_64], axis=1)
    z8_128 = jnp.zeros((64, 8), f32)
    w2e = jnp.concatenate([W2, jnp.dot(W2, as2, precision=hi), z8_128], axis=1)
    w2a = jnp.concatenate([jnp.dot(W2, ad2, precision=hi), z8_128], axis=1)
    gb = jnp.stack([gamma, beta], axis=0)

    p1 = jnp.concatenate([jnp.zeros((64, 64), f32),
                          jnp.kron(jnp.eye(8, dtype=f32), jnp.ones((1, 8), f32)),
                          jnp.zeros((8, 64), f32)], axis=0)
    p2 = jnp.concatenate([jnp.zeros((128, 128), f32),
                          jnp.kron(jnp.eye(8, dtype=f32), jnp.ones((1, 16), f32)),
                          jnp.zeros((8, 128), f32)], axis=0)
    smat = jnp.kron(jnp.ones((8, 1), f32), jnp.eye(16, dtype=f32))

    h1p, ad1p = pl.pallas_call(
        _tc1_body,
        out_shape=[jax.ShapeDtypeStruct((n, 80), f32),
                   jax.ShapeDtypeStruct((n, 16), f32)],
    )(x, gb, w1e, w1a)

    edge1 = _make_edge_kernel(n, e, heads=8, ch=8)
    acc1 = edge1(h1p, ad1p, src, dst, jnp.zeros((n, 80), f32))

    h2p, ad2p = pl.pallas_call(
        _tc2_body,
        out_shape=[jax.ShapeDtypeStruct((n, 144), f32),
                   jax.ShapeDtypeStruct((n, 16), f32)],
    )(acc1, b1.reshape(1, 64), p1, w2e, w2a)

    edge2 = _make_edge_kernel(n, e, heads=8, ch=16)
    acc2 = edge2(h2p, ad2p, src, dst, jnp.zeros((n, 144), f32))

    out = pl.pallas_call(
        _tc3_body,
        out_shape=jax.ShapeDtypeStruct((n, 16), f32),
    )(acc2, b2.reshape(1, 16), p2, smat)
    return out


# speedup vs baseline: 1.0005x; 1.0005x over previous
import functools

import jax
import jax.numpy as jnp
from jax import lax
from jax.experimental import pallas as pl
from jax.experimental.pallas import tpu as pltpu
from jax.experimental.pallas import tpu_sc as plsc

_NC = 2
_NS = 16
_LANES = 16


def _make_edge_kernel(n_nodes, n_edges, heads, ch):
    hc = heads * ch
    w = hc + 16
    nw = _NC * _NS
    ew = n_edges // nw
    assert ew * nw == n_edges
    chunk = 40
    nchunk = ew // chunk
    assert nchunk * chunk == ew
    rpt = n_nodes // _NS
    assert rpt * _NS == n_nodes
    groups = hc // _LANES
    hpg = _LANES // ch

    assert nchunk % 2 == 0 and nchunk >= 4
    mesh = plsc.VectorSubcoreMesh(core_axis_name="c", subcore_axis_name="s")

    def buf_scratch():
        return [
            pltpu.VMEM((chunk,), jnp.int32),
            pltpu.VMEM((chunk,), jnp.int32),
            pltpu.VMEM((chunk, w), jnp.float32),
            pltpu.VMEM((chunk, 16), jnp.float32),
            pltpu.VMEM((chunk, w), jnp.float32),
            pltpu.SemaphoreType.DMA,
            pltpu.SemaphoreType.DMA,
        ]

    @functools.partial(
        pl.kernel,
        out_type=jax.ShapeDtypeStruct((_NC, n_nodes, w), jnp.float32),
        mesh=mesh,
        compiler_params=pltpu.CompilerParams(use_tc_tiling_on_sc=False,
                                             needs_layout_passes=False),
        scratch_types=buf_scratch() + buf_scratch() + [
            pltpu.VMEM_SHARED((n_nodes, w), jnp.float32),
        ],
    )
    def edge_kernel(hsrc_hbm, ad_hbm, src_hbm, dst_hbm, zeros_hbm, out_hbm,
                    *scratch):
        bufs = (scratch[0:7], scratch[7:14])
        acc = scratch[14]
        c = lax.axis_index("c")
        s = lax.axis_index("s")
        wid = c * _NS + s
        pltpu.sync_copy(zeros_hbm.at[pl.ds(s * rpt, rpt)],
                        acc.at[pl.ds(s * rpt, rpt)])
        plsc.subcore_barrier()

        iota = lax.iota(jnp.int32, _LANES)
        gdn = lax.GatherDimensionNumbers(
            offset_dims=(), collapsed_slice_dims=(0,), start_index_map=(0,))

        def issue(ci, buf):
            sidx, didx, hrows, adrows, _, semg, _2 = buf
            base = wid * ew + ci * chunk
            pltpu.sync_copy(src_hbm.at[pl.ds(base, chunk)], sidx)
            pltpu.sync_copy(dst_hbm.at[pl.ds(base, chunk)], didx)
            pltpu.async_copy(hsrc_hbm.at[sidx], hrows, semg)
            pltpu.async_copy(ad_hbm.at[didx], adrows, semg)

        def scatter_start(buf):
            _, didx, _2, _3, msg, _4, sems = buf
            pltpu.async_copy(msg, acc.at[didx], sems, add=True)

        def scatter_wait(buf):
            _, didx, _2, _3, msg, _4, sems = buf
            pltpu.make_async_copy(msg, acc.at[didx], sems).wait()

        def compute(buf):
            sidx, didx, hrows, adrows, msg, semg, _ = buf
            pltpu.make_async_copy(hsrc_hbm.at[sidx], hrows, semg).wait()
            pltpu.make_async_copy(ad_hbm.at[didx], adrows, semg).wait()

            @plsc.parallel_loop(0, chunk, unroll=4)
            def edge_body(e):
                va = hrows[e, pl.ds(hc, 16)]
                vd = adrows[e, pl.ds(0, 16)]
                logit = va + vd
                logit = jnp.where(logit > 0, logit, 0.2 * logit)
                t = jnp.exp(logit)
                msg[e, pl.ds(hc, 16)] = t
                for g in range(groups):
                    gidx = g * hpg + iota // ch
                    mult = lax.gather(t, gidx[:, None], gdn, (1,),
                                      mode=lax.GatherScatterMode.PROMISE_IN_BOUNDS)
                    msg[e, pl.ds(g * _LANES, _LANES)] = (
                        hrows[e, pl.ds(g * _LANES, _LANES)] * mult)
        issue(0, bufs[0])
        issue(1, bufs[1])

        def pair_body(j, carry):
            c0 = 2 * j
            compute(bufs[0])
            scatter_start(bufs[0])
            compute(bufs[1])
            scatter_wait(bufs[0])
            issue(c0 + 2, bufs[0])
            scatter_start(bufs[1])
            scatter_wait(bufs[1])
            issue(c0 + 3, bufs[1])
            return carry

        lax.fori_loop(0, nchunk // 2 - 1, pair_body, 0)
        compute(bufs[0])
        scatter_start(bufs[0])
        compute(bufs[1])
        scatter_wait(bufs[0])
        scatter_start(bufs[1])
        scatter_wait(bufs[1])
        plsc.subcore_barrier()
        pltpu.sync_copy(acc.at[pl.ds(s * rpt, rpt)],
                        out_hbm.at[c, pl.ds(s * rpt, rpt)])

    return edge_kernel


def _tc1_body(x_ref, gb_ref, w1e_ref, w1a_ref, h_out, ad_out):
    x = x_ref[...]
    mean = jnp.mean(x, axis=0, keepdims=True)
    xc = x - mean
    var = jnp.mean(xc * xc, axis=0, keepdims=True)
    xh = xc / jnp.sqrt(var + 1e-5) * gb_ref[0:1, :] + gb_ref[1:2, :]
    h_out[...] = jnp.dot(xh, w1e_ref[...], preferred_element_type=jnp.float32)
    ad_out[...] = jnp.dot(xh, w1a_ref[...], preferred_element_type=jnp.float32)


def _tc2_body(acc_ref, b1_ref, p1_ref, w2e_ref, w2a_ref, h_out, ad_out):
    a = acc_ref[0] + acc_ref[1]
    num = a[:, :64]
    den = jnp.dot(a, p1_ref[...], preferred_element_type=jnp.float32,
                  precision=lax.Precision.HIGHEST)
    o = num / (den + 1e-16) + b1_ref[...]
    h = jnp.where(o > 0, o, jnp.exp(jnp.minimum(o, 0.0)) - 1.0)
    h_out[...] = jnp.dot(h, w2e_ref[...], preferred_element_type=jnp.float32)
    ad_out[...] = jnp.dot(h, w2a_ref[...], preferred_element_type=jnp.float32)


def _tc3_body(acc_ref, b2_ref, p2_ref, s_ref, out_ref):
    a = acc_ref[0] + acc_ref[1]
    num = a[:, :128]
    den = jnp.dot(a, p2_ref[...], preferred_element_type=jnp.float32,
                  precision=lax.Precision.HIGHEST)
    ratio = num / (den + 1e-16)
    o = jnp.dot(ratio, s_ref[...], preferred_element_type=jnp.float32,
                precision=lax.Precision.HIGHEST) * 0.125 + b2_ref[...]
    m = jnp.max(o, axis=1, keepdims=True)
    z = o - m
    lse = jnp.log(jnp.sum(jnp.exp(z), axis=1, keepdims=True))
    out_ref[...] = z - lse


def _head_proj(a_vec, heads, ch):
    return (a_vec[:, :, None] * jnp.eye(heads, dtype=a_vec.dtype)[:, None, :]
            ).reshape(heads * ch, heads)


def kernel(x, edge_index, gamma, beta, W1, a_src1, a_dst1, b1,
           W2, a_src2, a_dst2, b2):
    n, d = x.shape
    e = edge_index.shape[1]
    f32 = jnp.float32
    hi = lax.Precision.HIGHEST

    src = edge_index[0]
    dst = edge_index[1]

    as1 = _head_proj(a_src1, 8, 8)
    ad1 = _head_proj(a_dst1, 8, 8)
    as2 = _head_proj(a_src2, 8, 16)
    ad2 = _head_proj(a_dst2, 8, 16)
    z8_64 = jnp.zeros((d, 8), f32)
    w1e = jnp.concatenate([W1, jnp.dot(W1, as1, precision=hi), z8_64], axis=1)
    w1a = jnp.concatenate([jnp.dot(W1, ad1, precision=hi), z8_64], axis=1)
    z8_128 = jnp.zeros((64, 8), f32)
    w2e = jnp.concatenate([W2, jnp.dot(W2, as2, precision=hi), z8_128], axis=1)
    w2a = jnp.concatenate([jnp.dot(W2, ad2, precision=hi), z8_128], axis=1)
    gb = jnp.stack([gamma, beta], axis=0)

    p1 = jnp.concatenate([jnp.zeros((64, 64), f32),
                          jnp.kron(jnp.eye(8, dtype=f32), jnp.ones((1, 8), f32)),
                          jnp.zeros((8, 64), f32)], axis=0)
    p2 = jnp.concatenate([jnp.zeros((128, 128), f32),
                          jnp.kron(jnp.eye(8, dtype=f32), jnp.ones((1, 16), f32)),
                          jnp.zeros((8, 128), f32)], axis=0)
    smat = jnp.kron(jnp.ones((8, 1), f32), jnp.eye(16, dtype=f32))

    h1p, ad1p = pl.pallas_call(
        _tc1_body,
        out_shape=[jax.ShapeDtypeStruct((n, 80), f32),
                   jax.ShapeDtypeStruct((n, 16), f32)],
    )(x, gb, w1e, w1a)

    edge1 = _make_edge_kernel(n, e, heads=8, ch=8)
    acc1 = edge1(h1p, ad1p, src, dst, jnp.zeros((n, 80), f32))

    h2p, ad2p = pl.pallas_call(
        _tc2_body,
        out_shape=[jax.ShapeDtypeStruct((n, 144), f32),
                   jax.ShapeDtypeStruct((n, 16), f32)],
    )(acc1, b1.reshape(1, 64), p1, w2e, w2a)

    edge2 = _make_edge_kernel(n, e, heads=8, ch=16)
    acc2 = edge2(h2p, ad2p, src, dst, jnp.zeros((n, 144), f32))

    out = pl.pallas_call(
        _tc3_body,
        out_shape=jax.ShapeDtypeStruct((n, 16), f32),
    )(acc2, b2.reshape(1, 16), p2, smat)
    return out


# speedup vs baseline: 1.2942x; 1.2935x over previous
import functools

import jax
import jax.numpy as jnp
from jax import lax
from jax.experimental import pallas as pl
from jax.experimental.pallas import tpu as pltpu
from jax.experimental.pallas import tpu_sc as plsc

_NC = 2
_NS = 16
_LANES = 16


def _make_edge_kernel(n_nodes, n_edges, heads, ch):
    hc = heads * ch
    w = hc + 16
    nw = _NC * _NS
    ew = n_edges // nw
    assert ew * nw == n_edges
    def fits(ck):
        scratch_words = 16 * 2 * ck * (2 + w + 16 + w)
        return scratch_words + n_nodes * w <= 2_000_000
    chunk = 80 if fits(80) else 40
    nchunk = ew // chunk
    assert nchunk * chunk == ew
    rpt = n_nodes // _NS
    assert rpt * _NS == n_nodes
    groups = hc // _LANES
    hpg = _LANES // ch

    assert nchunk >= 4
    mesh = plsc.VectorSubcoreMesh(core_axis_name="c", subcore_axis_name="s")

    def buf_scratch():
        return [
            pltpu.VMEM((chunk,), jnp.int32),
            pltpu.VMEM((chunk,), jnp.int32),
            pltpu.VMEM((chunk, w), jnp.float32),
            pltpu.VMEM((chunk, 16), jnp.float32),
            pltpu.VMEM((chunk, w), jnp.float32),
            pltpu.SemaphoreType.DMA,
        ]

    @functools.partial(
        pl.kernel,
        out_type=jax.ShapeDtypeStruct((_NC, n_nodes, w), jnp.float32),
        mesh=mesh,
        compiler_params=pltpu.CompilerParams(use_tc_tiling_on_sc=False,
                                             needs_layout_passes=False),
        scratch_types=buf_scratch() + buf_scratch() + [
            pltpu.VMEM_SHARED((n_nodes, w), jnp.float32),
        ],
    )
    def edge_kernel(hsrc_hbm, ad_hbm, src_hbm, dst_hbm, zeros_hbm, out_hbm,
                    *scratch):
        bufs = (scratch[0:6], scratch[6:12])
        acc = scratch[12]
        c = lax.axis_index("c")
        s = lax.axis_index("s")
        wid = c * _NS + s
        pltpu.sync_copy(zeros_hbm.at[pl.ds(s * rpt, rpt)],
                        acc.at[pl.ds(s * rpt, rpt)])
        plsc.subcore_barrier()

        iota = lax.iota(jnp.int32, _LANES)
        gdn = lax.GatherDimensionNumbers(
            offset_dims=(), collapsed_slice_dims=(0,), start_index_map=(0,))

        def issue(ci, buf):
            sidx, didx, hrows, adrows, _, semg = buf
            base = wid * ew + ci * chunk
            pltpu.sync_copy(src_hbm.at[pl.ds(base, chunk)], sidx)
            pltpu.sync_copy(dst_hbm.at[pl.ds(base, chunk)], didx)
            pltpu.async_copy(hsrc_hbm.at[sidx], hrows, semg)
            pltpu.async_copy(ad_hbm.at[didx], adrows, semg)

        def process(buf):
            sidx, didx, hrows, adrows, msg, semg = buf
            pltpu.make_async_copy(hsrc_hbm.at[sidx], hrows, semg).wait()
            pltpu.make_async_copy(ad_hbm.at[didx], adrows, semg).wait()

            @plsc.parallel_loop(0, chunk, unroll=4)
            def edge_body(e):
                va = hrows[e, pl.ds(hc, 16)]
                vd = adrows[e, pl.ds(0, 16)]
                logit = va + vd
                logit = jnp.where(logit > 0, logit, 0.2 * logit)
                t = jnp.exp(logit)
                msg[e, pl.ds(hc, 16)] = t
                for g in range(groups):
                    gidx = g * hpg + iota // ch
                    mult = lax.gather(t, gidx[:, None], gdn, (1,),
                                      mode=lax.GatherScatterMode.PROMISE_IN_BOUNDS)
                    msg[e, pl.ds(g * _LANES, _LANES)] = (
                        hrows[e, pl.ds(g * _LANES, _LANES)] * mult)
            pltpu.sync_copy(msg, acc.at[didx], add=True)

        issue(0, bufs[0])
        issue(1, bufs[1])

        def pair_body(j, carry):
            c0 = 2 * j
            process(bufs[0])
            issue(c0 + 2, bufs[0])
            process(bufs[1])

            @pl.when(c0 + 3 < nchunk)
            def _():
                issue(c0 + 3, bufs[1])
            return carry

        lax.fori_loop(0, (nchunk - 1) // 2, pair_body, 0)
        process(bufs[0])
        if nchunk % 2 == 0:
            process(bufs[1])
        plsc.subcore_barrier()
        pltpu.sync_copy(acc.at[pl.ds(s * rpt, rpt)],
                        out_hbm.at[c, pl.ds(s * rpt, rpt)])

    return edge_kernel


def _tc1_body(x_ref, gb_ref, w1e_ref, w1a_ref, h_out, ad_out):
    x = x_ref[...]
    mean = jnp.mean(x, axis=0, keepdims=True)
    xc = x - mean
    var = jnp.mean(xc * xc, axis=0, keepdims=True)
    xh = xc / jnp.sqrt(var + 1e-5) * gb_ref[0:1, :] + gb_ref[1:2, :]
    h_out[...] = jnp.dot(xh, w1e_ref[...], preferred_element_type=jnp.float32)
    ad_out[...] = jnp.dot(xh, w1a_ref[...], preferred_element_type=jnp.float32)


def _tc2_body(acc_ref, b1_ref, p1_ref, w2e_ref, w2a_ref, h_out, ad_out):
    a = acc_ref[0] + acc_ref[1]
    num = a[:, :64]
    den = jnp.dot(a, p1_ref[...], preferred_element_type=jnp.float32,
                  precision=lax.Precision.HIGHEST)
    o = num / (den + 1e-16) + b1_ref[...]
    h = jnp.where(o > 0, o, jnp.exp(jnp.minimum(o, 0.0)) - 1.0)
    h_out[...] = jnp.dot(h, w2e_ref[...], preferred_element_type=jnp.float32)
    ad_out[...] = jnp.dot(h, w2a_ref[...], preferred_element_type=jnp.float32)


def _tc3_body(acc_ref, b2_ref, p2_ref, s_ref, out_ref):
    a = acc_ref[0] + acc_ref[1]
    num = a[:, :128]
    den = jnp.dot(a, p2_ref[...], preferred_element_type=jnp.float32,
                  precision=lax.Precision.HIGHEST)
    ratio = num / (den + 1e-16)
    o = jnp.dot(ratio, s_ref[...], preferred_element_type=jnp.float32,
                precision=lax.Precision.HIGHEST) * 0.125 + b2_ref[...]
    m = jnp.max(o, axis=1, keepdims=True)
    z = o - m
    lse = jnp.log(jnp.sum(jnp.exp(z), axis=1, keepdims=True))
    out_ref[...] = z - lse


def _head_proj(a_vec, heads, ch):
    return (a_vec[:, :, None] * jnp.eye(heads, dtype=a_vec.dtype)[:, None, :]
            ).reshape(heads * ch, heads)


def kernel(x, edge_index, gamma, beta, W1, a_src1, a_dst1, b1,
           W2, a_src2, a_dst2, b2):
    n, d = x.shape
    e = edge_index.shape[1]
    f32 = jnp.float32
    hi = lax.Precision.HIGHEST

    src = edge_index[0]
    dst = edge_index[1]

    as1 = _head_proj(a_src1, 8, 8)
    ad1 = _head_proj(a_dst1, 8, 8)
    as2 = _head_proj(a_src2, 8, 16)
    ad2 = _head_proj(a_dst2, 8, 16)
    z8_64 = jnp.zeros((d, 8), f32)
    w1e = jnp.concatenate([W1, jnp.dot(W1, as1, precision=hi), z8_64], axis=1)
    w1a = jnp.concatenate([jnp.dot(W1, ad1, precision=hi), z8_64], axis=1)
    z8_128 = jnp.zeros((64, 8), f32)
    w2e = jnp.concatenate([W2, jnp.dot(W2, as2, precision=hi), z8_128], axis=1)
    w2a = jnp.concatenate([jnp.dot(W2, ad2, precision=hi), z8_128], axis=1)
    gb = jnp.stack([gamma, beta], axis=0)

    p1 = jnp.concatenate([jnp.zeros((64, 64), f32),
                          jnp.kron(jnp.eye(8, dtype=f32), jnp.ones((1, 8), f32)),
                          jnp.zeros((8, 64), f32)], axis=0)
    p2 = jnp.concatenate([jnp.zeros((128, 128), f32),
                          jnp.kron(jnp.eye(8, dtype=f32), jnp.ones((1, 16), f32)),
                          jnp.zeros((8, 128), f32)], axis=0)
    smat = jnp.kron(jnp.ones((8, 1), f32), jnp.eye(16, dtype=f32))

    h1p, ad1p = pl.pallas_call(
        _tc1_body,
        out_shape=[jax.ShapeDtypeStruct((n, 80), f32),
                   jax.ShapeDtypeStruct((n, 16), f32)],
    )(x, gb, w1e, w1a)

    edge1 = _make_edge_kernel(n, e, heads=8, ch=8)
    acc1 = edge1(h1p, ad1p, src, dst, jnp.zeros((n, 80), f32))

    h2p, ad2p = pl.pallas_call(
        _tc2_body,
        out_shape=[jax.ShapeDtypeStruct((n, 144), f32),
                   jax.ShapeDtypeStruct((n, 16), f32)],
    )(acc1, b1.reshape(1, 64), p1, w2e, w2a)

    edge2 = _make_edge_kernel(n, e, heads=8, ch=16)
    acc2 = edge2(h2p, ad2p, src, dst, jnp.zeros((n, 144), f32))

    out = pl.pallas_call(
        _tc3_body,
        out_shape=jax.ShapeDtypeStruct((n, 16), f32),
    )(acc2, b2.reshape(1, 16), p2, smat)
    return out


# speedup vs baseline: 1.5351x; 1.1861x over previous
import functools

import jax
import jax.numpy as jnp
from jax import lax
from jax.experimental import pallas as pl
from jax.experimental.pallas import tpu as pltpu
from jax.experimental.pallas import tpu_sc as plsc

_NC = 2
_NS = 16
_LANES = 16


def _make_edge_kernel(n_nodes, n_edges, heads, ch):
    hc = heads * ch
    w = hc + 16
    nw = _NC * _NS
    ew = n_edges // nw
    assert ew * nw == n_edges
    def fits(ck):
        scratch_words = 16 * ck * (2 * (2 + w + 16) + w)
        return scratch_words + n_nodes * w <= 2_090_000
    chunk = 80 if fits(80) else 40
    nchunk = ew // chunk
    assert nchunk * chunk == ew
    rpt = n_nodes // _NS
    assert rpt * _NS == n_nodes
    groups = hc // _LANES
    hpg = _LANES // ch

    assert nchunk >= 4
    mesh = plsc.VectorSubcoreMesh(core_axis_name="c", subcore_axis_name="s")

    def buf_scratch():
        return [
            pltpu.VMEM((chunk,), jnp.int32),
            pltpu.VMEM((chunk,), jnp.int32),
            pltpu.VMEM((chunk, w), jnp.float32),
            pltpu.VMEM((chunk, 16), jnp.float32),
            pltpu.SemaphoreType.DMA,
        ]

    @functools.partial(
        pl.kernel,
        out_type=jax.ShapeDtypeStruct((_NC, n_nodes, w), jnp.float32),
        mesh=mesh,
        compiler_params=pltpu.CompilerParams(use_tc_tiling_on_sc=False,
                                             needs_layout_passes=False),
        scratch_types=buf_scratch() + buf_scratch() + [
            pltpu.VMEM((chunk, w), jnp.float32),
            pltpu.VMEM_SHARED((n_nodes, w), jnp.float32),
        ],
    )
    def edge_kernel(hsrc_hbm, ad_hbm, src_hbm, dst_hbm, zeros_hbm, out_hbm,
                    *scratch):
        bufs = (scratch[0:5], scratch[5:10])
        msg = scratch[10]
        acc = scratch[11]
        c = lax.axis_index("c")
        s = lax.axis_index("s")
        wid = c * _NS + s
        pltpu.sync_copy(zeros_hbm.at[pl.ds(s * rpt, rpt)],
                        acc.at[pl.ds(s * rpt, rpt)])
        plsc.subcore_barrier()

        iota = lax.iota(jnp.int32, _LANES)
        gdn = lax.GatherDimensionNumbers(
            offset_dims=(), collapsed_slice_dims=(0,), start_index_map=(0,))

        def issue(ci, buf):
            sidx, didx, hrows, adrows, semg = buf
            base = wid * ew + ci * chunk
            pltpu.sync_copy(src_hbm.at[pl.ds(base, chunk)], sidx)
            pltpu.sync_copy(dst_hbm.at[pl.ds(base, chunk)], didx)
            pltpu.async_copy(hsrc_hbm.at[sidx], hrows, semg)
            pltpu.async_copy(ad_hbm.at[didx], adrows, semg)

        def process(buf):
            sidx, didx, hrows, adrows, semg = buf
            pltpu.make_async_copy(hsrc_hbm.at[sidx], hrows, semg).wait()
            pltpu.make_async_copy(ad_hbm.at[didx], adrows, semg).wait()

            @plsc.parallel_loop(0, chunk, unroll=4)
            def edge_body(e):
                va = hrows[e, pl.ds(hc, 16)]
                vd = adrows[e, pl.ds(0, 16)]
                logit = va + vd
                logit = jnp.where(logit > 0, logit, 0.2 * logit)
                t = jnp.exp(logit)
                msg[e, pl.ds(hc, 16)] = t
                for g in range(groups):
                    gidx = g * hpg + iota // ch
                    mult = lax.gather(t, gidx[:, None], gdn, (1,),
                                      mode=lax.GatherScatterMode.PROMISE_IN_BOUNDS)
                    msg[e, pl.ds(g * _LANES, _LANES)] = (
                        hrows[e, pl.ds(g * _LANES, _LANES)] * mult)
            pltpu.sync_copy(msg, acc.at[didx], add=True)

        issue(0, bufs[0])
        issue(1, bufs[1])

        def pair_body(j, carry):
            c0 = 2 * j
            process(bufs[0])
            issue(c0 + 2, bufs[0])
            process(bufs[1])

            @pl.when(c0 + 3 < nchunk)
            def _():
                issue(c0 + 3, bufs[1])
            return carry

        lax.fori_loop(0, (nchunk - 1) // 2, pair_body, 0)
        process(bufs[0])
        if nchunk % 2 == 0:
            process(bufs[1])
        plsc.subcore_barrier()
        pltpu.sync_copy(acc.at[pl.ds(s * rpt, rpt)],
                        out_hbm.at[c, pl.ds(s * rpt, rpt)])

    return edge_kernel


def _tc1_body(x_ref, gb_ref, w1e_ref, w1a_ref, h_out, ad_out):
    x = x_ref[...]
    mean = jnp.mean(x, axis=0, keepdims=True)
    xc = x - mean
    var = jnp.mean(xc * xc, axis=0, keepdims=True)
    xh = xc / jnp.sqrt(var + 1e-5) * gb_ref[0:1, :] + gb_ref[1:2, :]
    h_out[...] = jnp.dot(xh, w1e_ref[...], preferred_element_type=jnp.float32)
    ad_out[...] = jnp.dot(xh, w1a_ref[...], preferred_element_type=jnp.float32)


def _tc2_body(acc_ref, b1_ref, p1_ref, w2e_ref, w2a_ref, h_out, ad_out):
    a = acc_ref[0] + acc_ref[1]
    num = a[:, :64]
    den = jnp.dot(a, p1_ref[...], preferred_element_type=jnp.float32,
                  precision=lax.Precision.HIGHEST)
    o = num / (den + 1e-16) + b1_ref[...]
    h = jnp.where(o > 0, o, jnp.exp(jnp.minimum(o, 0.0)) - 1.0)
    h_out[...] = jnp.dot(h, w2e_ref[...], preferred_element_type=jnp.float32)
    ad_out[...] = jnp.dot(h, w2a_ref[...], preferred_element_type=jnp.float32)


def _tc3_body(acc_ref, b2_ref, p2_ref, s_ref, out_ref):
    a = acc_ref[0] + acc_ref[1]
    num = a[:, :128]
    den = jnp.dot(a, p2_ref[...], preferred_element_type=jnp.float32,
                  precision=lax.Precision.HIGHEST)
    ratio = num / (den + 1e-16)
    o = jnp.dot(ratio, s_ref[...], preferred_element_type=jnp.float32,
                precision=lax.Precision.HIGHEST) * 0.125 + b2_ref[...]
    m = jnp.max(o, axis=1, keepdims=True)
    z = o - m
    lse = jnp.log(jnp.sum(jnp.exp(z), axis=1, keepdims=True))
    out_ref[...] = z - lse


def _head_proj(a_vec, heads, ch):
    return (a_vec[:, :, None] * jnp.eye(heads, dtype=a_vec.dtype)[:, None, :]
            ).reshape(heads * ch, heads)


def kernel(x, edge_index, gamma, beta, W1, a_src1, a_dst1, b1,
           W2, a_src2, a_dst2, b2):
    n, d = x.shape
    e = edge_index.shape[1]
    f32 = jnp.float32
    hi = lax.Precision.HIGHEST

    src = edge_index[0]
    dst = edge_index[1]

    as1 = _head_proj(a_src1, 8, 8)
    ad1 = _head_proj(a_dst1, 8, 8)
    as2 = _head_proj(a_src2, 8, 16)
    ad2 = _head_proj(a_dst2, 8, 16)
    z8_64 = jnp.zeros((d, 8), f32)
    w1e = jnp.concatenate([W1, jnp.dot(W1, as1, precision=hi), z8_64], axis=1)
    w1a = jnp.concatenate([jnp.dot(W1, ad1, precision=hi), z8_64], axis=1)
    z8_128 = jnp.zeros((64, 8), f32)
    w2e = jnp.concatenate([W2, jnp.dot(W2, as2, precision=hi), z8_128], axis=1)
    w2a = jnp.concatenate([jnp.dot(W2, ad2, precision=hi), z8_128], axis=1)
    gb = jnp.stack([gamma, beta], axis=0)

    p1 = jnp.concatenate([jnp.zeros((64, 64), f32),
                          jnp.kron(jnp.eye(8, dtype=f32), jnp.ones((1, 8), f32)),
                          jnp.zeros((8, 64), f32)], axis=0)
    p2 = jnp.concatenate([jnp.zeros((128, 128), f32),
                          jnp.kron(jnp.eye(8, dtype=f32), jnp.ones((1, 16), f32)),
                          jnp.zeros((8, 128), f32)], axis=0)
    smat = jnp.kron(jnp.ones((8, 1), f32), jnp.eye(16, dtype=f32))

    h1p, ad1p = pl.pallas_call(
        _tc1_body,
        out_shape=[jax.ShapeDtypeStruct((n, 80), f32),
                   jax.ShapeDtypeStruct((n, 16), f32)],
    )(x, gb, w1e, w1a)

    edge1 = _make_edge_kernel(n, e, heads=8, ch=8)
    acc1 = edge1(h1p, ad1p, src, dst, jnp.zeros((n, 80), f32))

    h2p, ad2p = pl.pallas_call(
        _tc2_body,
        out_shape=[jax.ShapeDtypeStruct((n, 144), f32),
                   jax.ShapeDtypeStruct((n, 16), f32)],
    )(acc1, b1.reshape(1, 64), p1, w2e, w2a)

    edge2 = _make_edge_kernel(n, e, heads=8, ch=16)
    acc2 = edge2(h2p, ad2p, src, dst, jnp.zeros((n, 144), f32))

    out = pl.pallas_call(
        _tc3_body,
        out_shape=jax.ShapeDtypeStruct((n, 16), f32),
    )(acc2, b2.reshape(1, 16), p2, smat)
    return out
